# baseline (speedup 1.0000x reference)
import functools

import jax
import jax.numpy as jnp
from jax import lax
from jax.experimental import pallas as pl
from jax.experimental.pallas import tpu as pltpu
from jax.experimental.pallas import tpu_sc as plsc

VOCAB = 1000000
DIM = 64
BATCH = 16384
CHUNK = 128


@functools.cache
def _make_fused(V, D, B):
    info = plsc.get_sparse_core_info()
    NC, NS = info.num_cores, info.num_subcores
    NW = NC * NS
    assert B % NW == 0
    b_per_w = B // NW
    assert b_per_w % CHUNK == 0
    n_chunks = b_per_w // CHUNK
    assert V % NW == 0
    v_per_w = V // NW
    bias_per_w = ((v_per_w + 7) // 8) * 8
    mesh = plsc.VectorSubcoreMesh(core_axis_name="c", subcore_axis_name="s")

    @functools.partial(
        pl.kernel,
        mesh=mesh,
        compiler_params=pltpu.CompilerParams(use_tc_tiling_on_sc=False),
        out_type=(
            jax.ShapeDtypeStruct((B, D), jnp.float32),
            jax.ShapeDtypeStruct((V, D), jnp.float32),
            jax.ShapeDtypeStruct((V,), jnp.float32),
        ),
        scratch_types=[
            pltpu.VMEM((n_chunks, CHUNK), jnp.int32),
            pltpu.VMEM((b_per_w, D), jnp.float32),
            pltpu.SemaphoreType.DMA,
            pltpu.SemaphoreType.DMA,
        ],
    )
    def fused_kernel(idx_hbm, table_hbm, w_hbm, b_hbm,
                     out_e, out_w, out_b, idx_v, rows_v, sem, csem):
        wid = lax.axis_index("s") * NC + lax.axis_index("c")
        base = wid * b_per_w
        pltpu.sync_copy(idx_hbm.at[wid], idx_v)
        copies = [
            pltpu.async_copy(
                table_hbm.at[idx_v.at[j]],
                rows_v.at[pl.ds(j * CHUNK, CHUNK)],
                sem,
            )
            for j in range(n_chunks)
        ]
        wbase = wid * v_per_w
        wcopy = pltpu.async_copy(
            w_hbm.at[pl.ds(wbase, v_per_w)],
            out_w.at[pl.ds(wbase, v_per_w)],
            csem,
        )
        bbase = jnp.minimum(wid * bias_per_w, V - bias_per_w)
        bcopy = pltpu.async_copy(
            b_hbm.at[pl.ds(bbase, bias_per_w)],
            out_b.at[pl.ds(bbase, bias_per_w)],
            csem,
        )
        for c in copies:
            c.wait()
        pltpu.sync_copy(rows_v, out_e.at[pl.ds(base, b_per_w)])
        wcopy.wait()
        bcopy.wait()

    return fused_kernel


def kernel(inputs, embedding_table, nce_weights, nce_biases):
    info = plsc.get_sparse_core_info()
    NW = info.num_cores * info.num_subcores
    idx3 = inputs.reshape(NW, BATCH // NW // CHUNK, CHUNK)
    embed, w, b = _make_fused(VOCAB, DIM, BATCH)(
        idx3, embedding_table, nce_weights, nce_biases)
    return (embed, w, b)


# speedup vs baseline: 5.0658x; 5.0658x over previous
import functools

import jax
import jax.numpy as jnp
from jax import lax
from jax.experimental import pallas as pl
from jax.experimental.pallas import tpu as pltpu
from jax.experimental.pallas import tpu_sc as plsc

VOCAB = 1000000
DIM = 64
BATCH = 16384
CHUNK = 128
WCHUNK = 40000
NBUF = 2


@functools.cache
def _make_fused(V, D, B):
    info = plsc.get_sparse_core_info()
    NC, NS = info.num_cores, info.num_subcores
    NW = NC * NS
    b_per_w = B // NW
    n_chunks = b_per_w // CHUNK
    w_elems = V * D // NW
    n_wchunks = w_elems // WCHUNK
    n_rounds = n_wchunks // NBUF
    assert n_rounds * NBUF * WCHUNK == w_elems
    bias_per_w = ((V // NW + 7) // 8) * 8
    assert bias_per_w <= WCHUNK
    mesh = plsc.VectorSubcoreMesh(core_axis_name="c", subcore_axis_name="s")

    @functools.partial(
        pl.kernel,
        mesh=mesh,
        compiler_params=pltpu.CompilerParams(use_tc_tiling_on_sc=False),
        out_type=(
            jax.ShapeDtypeStruct((B, D), jnp.float32),
            jax.ShapeDtypeStruct((V * D,), jnp.float32),
            jax.ShapeDtypeStruct((V,), jnp.float32),
        ),
        scratch_types=[
            pltpu.VMEM((n_chunks, CHUNK), jnp.int32),
            pltpu.VMEM((b_per_w, D), jnp.float32),
            pltpu.VMEM((NBUF, WCHUNK), jnp.float32),
            pltpu.SemaphoreType.DMA,
            pltpu.SemaphoreType.DMA,
            pltpu.SemaphoreType.DMA,
            pltpu.SemaphoreType.DMA,
            pltpu.SemaphoreType.DMA,
        ],
    )
    def fused_kernel(idx_hbm, table_hbm, w_hbm, b_hbm,
                     out_e, out_w, out_b,
                     idx_v, rows_v, buf, gsem, rs0, rs1, ws0, ws1):
        rs = [rs0, rs1]
        ws = [ws0, ws1]
        wid = lax.axis_index("s") * NC + lax.axis_index("c")
        base = wid * b_per_w

        pltpu.sync_copy(idx_hbm.at[wid], idx_v)
        gathers = [
            pltpu.async_copy(
                table_hbm.at[idx_v.at[j]],
                rows_v.at[pl.ds(j * CHUNK, CHUNK)],
                gsem,
            )
            for j in range(n_chunks)
        ]

        wb = wid * w_elems
        for b in range(NBUF):
            pltpu.async_copy(
                w_hbm.at[pl.ds(wb + b * WCHUNK, WCHUNK)], buf.at[b], rs[b])

        def round_body(r, _):
            off = wb + r * (NBUF * WCHUNK)
            for b in range(NBUF):
                src = w_hbm.at[pl.ds(off + b * WCHUNK, WCHUNK)]
                dst = out_w.at[pl.ds(off + b * WCHUNK, WCHUNK)]
                pltpu.make_async_copy(src, buf.at[b], rs[b]).wait()
                pltpu.async_copy(buf.at[b], dst, ws[b])
            nxt = off + NBUF * WCHUNK

            @pl.when(r < n_rounds - 1)
            def _():
                for b in range(NBUF):
                    pltpu.make_async_copy(
                        buf.at[b],
                        out_w.at[pl.ds(nxt + b * WCHUNK - NBUF * WCHUNK, WCHUNK)],
                        ws[b],
                    ).wait()
                    pltpu.async_copy(
                        w_hbm.at[pl.ds(nxt + b * WCHUNK, WCHUNK)],
                        buf.at[b], rs[b])
            return 0

        lax.fori_loop(0, n_rounds, round_body, 0)

        last = wb + (n_rounds - 1) * NBUF * WCHUNK
        for b in range(NBUF):
            pltpu.make_async_copy(
                buf.at[b], out_w.at[pl.ds(last + b * WCHUNK, WCHUNK)], ws[b]
            ).wait()

        bb = jnp.minimum(wid * bias_per_w, V - bias_per_w)
        pltpu.sync_copy(b_hbm.at[pl.ds(bb, bias_per_w)],
                        buf.at[0, pl.ds(0, bias_per_w)])
        pltpu.sync_copy(buf.at[0, pl.ds(0, bias_per_w)],
                        out_b.at[pl.ds(bb, bias_per_w)])

        for g in gathers:
            g.wait()
        pltpu.sync_copy(rows_v, out_e.at[pl.ds(base, b_per_w)])

    return fused_kernel


def kernel(inputs, embedding_table, nce_weights, nce_biases):
    info = plsc.get_sparse_core_info()
    NW = info.num_cores * info.num_subcores
    idx3 = inputs.reshape(NW, BATCH // NW // CHUNK, CHUNK)
    embed, w_flat, b = _make_fused(VOCAB, DIM, BATCH)(
        idx3, embedding_table, nce_weights.reshape(-1), nce_biases)
    return (embed, w_flat.reshape(VOCAB, DIM), b)


# speedup vs baseline: 5.0685x; 1.0005x over previous
import functools

import jax
import jax.numpy as jnp
from jax import lax
from jax.experimental import pallas as pl
from jax.experimental.pallas import tpu as pltpu
from jax.experimental.pallas import tpu_sc as plsc

VOCAB = 1000000
DIM = 64
BATCH = 16384
CHUNK = 128
WROWS = 512
NBUF = 2


@functools.cache
def _make_fused(V, D, B):
    info = plsc.get_sparse_core_info()
    NC, NS = info.num_cores, info.num_subcores
    NW = NC * NS
    b_per_w = B // NW
    n_chunks = b_per_w // CHUNK
    w_rows = V // NW
    n_wchunks = -(-w_rows // WROWS)
    n_rounds = -(-n_wchunks // NBUF)
    bias_per_w = ((V // NW + 7) // 8) * 8
    mesh = plsc.VectorSubcoreMesh(core_axis_name="c", subcore_axis_name="s")

    @functools.partial(
        pl.kernel,
        mesh=mesh,
        compiler_params=pltpu.CompilerParams(use_tc_tiling_on_sc=False),
        out_type=(
            jax.ShapeDtypeStruct((B, D), jnp.float32),
            jax.ShapeDtypeStruct((V, D), jnp.float32),
            jax.ShapeDtypeStruct((V,), jnp.float32),
        ),
        scratch_types=[
            pltpu.VMEM((n_chunks, CHUNK), jnp.int32),
            pltpu.VMEM((b_per_w, D), jnp.float32),
            pltpu.VMEM((NBUF, WROWS, D), jnp.float32),
            pltpu.VMEM((bias_per_w,), jnp.float32),
            pltpu.SemaphoreType.DMA,
            pltpu.SemaphoreType.DMA,
            pltpu.SemaphoreType.DMA,
            pltpu.SemaphoreType.DMA,
            pltpu.SemaphoreType.DMA,
        ],
    )
    def fused_kernel(idx_hbm, table_hbm, w_hbm, b_hbm,
                     out_e, out_w, out_b,
                     idx_v, rows_v, buf, bias_v, gsem, rs0, rs1, ws0, ws1):
        rs = [rs0, rs1]
        ws = [ws0, ws1]
        wid = lax.axis_index("s") * NC + lax.axis_index("c")
        base = wid * b_per_w

        pltpu.sync_copy(idx_hbm.at[wid], idx_v)
        gathers = [
            pltpu.async_copy(
                table_hbm.at[idx_v.at[j]],
                rows_v.at[pl.ds(j * CHUNK, CHUNK)],
                gsem,
            )
            for j in range(n_chunks)
        ]

        wb = wid * w_rows

        def chunk_off(c):
            return wb + jnp.minimum(c * WROWS, w_rows - WROWS)

        for b in range(NBUF):
            pltpu.async_copy(
                w_hbm.at[pl.ds(chunk_off(b), WROWS)], buf.at[b], rs[b])

        def round_body(r, _):
            for b in range(NBUF):
                c = r * NBUF + b
                off = chunk_off(c)
                pltpu.make_async_copy(
                    w_hbm.at[pl.ds(off, WROWS)], buf.at[b], rs[b]).wait()
                pltpu.async_copy(
                    buf.at[b], out_w.at[pl.ds(off, WROWS)], ws[b])

            @pl.when(r < n_rounds - 1)
            def _():
                for b in range(NBUF):
                    c = r * NBUF + b
                    pltpu.make_async_copy(
                        buf.at[b],
                        out_w.at[pl.ds(chunk_off(c), WROWS)],
                        ws[b],
                    ).wait()
                    pltpu.async_copy(
                        w_hbm.at[pl.ds(chunk_off(c + NBUF), WROWS)],
                        buf.at[b], rs[b])
            return 0

        lax.fori_loop(0, n_rounds, round_body, 0)

        for b in range(NBUF):
            c = (n_rounds - 1) * NBUF + b
            pltpu.make_async_copy(
                buf.at[b], out_w.at[pl.ds(chunk_off(c), WROWS)], ws[b]
            ).wait()

        bb = jnp.minimum(wid * bias_per_w, V - bias_per_w)
        pltpu.sync_copy(b_hbm.at[pl.ds(bb, bias_per_w)], bias_v)
        pltpu.sync_copy(bias_v, out_b.at[pl.ds(bb, bias_per_w)])

        for g in gathers:
            g.wait()
        pltpu.sync_copy(rows_v, out_e.at[pl.ds(base, b_per_w)])

    return fused_kernel


def kernel(inputs, embedding_table, nce_weights, nce_biases):
    info = plsc.get_sparse_core_info()
    NW = info.num_cores * info.num_subcores
    idx3 = inputs.reshape(NW, BATCH // NW // CHUNK, CHUNK)
    embed, w, b = _make_fused(VOCAB, DIM, BATCH)(
        idx3, embedding_table, nce_weights, nce_biases)
    return (embed, w, b)


# speedup vs baseline: 9.6750x; 1.9088x over previous
import functools

import jax
import jax.numpy as jnp
from jax import lax
from jax.experimental import pallas as pl
from jax.experimental.pallas import tpu as pltpu
from jax.experimental.pallas import tpu_sc as plsc

VOCAB = 1000000
DIM = 64
BATCH = 16384
CHUNK = 128
WBLK = 2048


@functools.cache
def _make_gather(V, D, B):
    info = plsc.get_sparse_core_info()
    NC, NS = info.num_cores, info.num_subcores
    NW = NC * NS
    b_per_w = B // NW
    n_chunks = b_per_w // CHUNK
    mesh = plsc.VectorSubcoreMesh(core_axis_name="c", subcore_axis_name="s")

    @functools.partial(
        pl.kernel,
        mesh=mesh,
        compiler_params=pltpu.CompilerParams(use_tc_tiling_on_sc=False),
        out_type=jax.ShapeDtypeStruct((B, D), jnp.float32),
        scratch_types=[
            pltpu.VMEM((n_chunks, CHUNK), jnp.int32),
            pltpu.VMEM((b_per_w, D), jnp.float32),
            pltpu.SemaphoreType.DMA,
        ],
    )
    def gather_kernel(idx_hbm, table_hbm, out_e, idx_v, rows_v, gsem):
        wid = lax.axis_index("s") * NC + lax.axis_index("c")
        base = wid * b_per_w

        pltpu.sync_copy(idx_hbm.at[wid], idx_v)
        gathers = [
            pltpu.async_copy(
                table_hbm.at[idx_v.at[j]],
                rows_v.at[pl.ds(j * CHUNK, CHUNK)],
                gsem,
            )
            for j in range(n_chunks)
        ]
        for g in gathers:
            g.wait()
        pltpu.sync_copy(rows_v, out_e.at[pl.ds(base, b_per_w)])

    return gather_kernel


def _copy_block(src_ref, dst_ref):
    dst_ref[...] = src_ref[...]


@functools.cache
def _make_wcopy(V, D):
    n_blocks = -(-V // WBLK)
    return pl.pallas_call(
        _copy_block,
        grid=(n_blocks,),
        in_specs=[pl.BlockSpec((D, WBLK), lambda i: (0, i))],
        out_specs=pl.BlockSpec((D, WBLK), lambda i: (0, i)),
        out_shape=jax.ShapeDtypeStruct((D, V), jnp.float32),
    )


def kernel(inputs, embedding_table, nce_weights, nce_biases):
    info = plsc.get_sparse_core_info()
    NW = info.num_cores * info.num_subcores
    idx3 = inputs.reshape(NW, BATCH // NW // CHUNK, CHUNK)
    embed = _make_gather(VOCAB, DIM, BATCH)(idx3, embedding_table)
    w_t = _make_wcopy(VOCAB, DIM)(nce_weights.T)
    return (embed, w_t.T, nce_biases)


# speedup vs baseline: 12.6080x; 1.3032x over previous
import functools

import jax
import jax.numpy as jnp
from jax import lax
from jax.experimental import pallas as pl
from jax.experimental.pallas import tpu as pltpu
from jax.experimental.pallas import tpu_sc as plsc

VOCAB = 1000000
DIM = 64
BATCH = 16384
CHUNK = 128
WBLK = 8192


@functools.cache
def _make_gather(V, D, B):
    info = plsc.get_sparse_core_info()
    NC, NS = info.num_cores, info.num_subcores
    NW = NC * NS
    b_per_w = B // NW
    n_chunks = b_per_w // CHUNK
    mesh = plsc.VectorSubcoreMesh(core_axis_name="c", subcore_axis_name="s")

    @functools.partial(
        pl.kernel,
        mesh=mesh,
        compiler_params=pltpu.CompilerParams(use_tc_tiling_on_sc=False),
        out_type=jax.ShapeDtypeStruct((B, D), jnp.float32),
        scratch_types=[
            pltpu.VMEM((n_chunks, CHUNK), jnp.int32),
            pltpu.VMEM((b_per_w, D), jnp.float32),
            pltpu.SemaphoreType.DMA,
        ],
    )
    def gather_kernel(idx_hbm, table_hbm, dummy_hbm, out_e, idx_v, rows_v,
                      gsem):
        del dummy_hbm
        wid = lax.axis_index("s") * NC + lax.axis_index("c")
        base = wid * b_per_w

        pltpu.sync_copy(idx_hbm.at[wid], idx_v)
        gathers = [
            pltpu.async_copy(
                table_hbm.at[idx_v.at[j]],
                rows_v.at[pl.ds(j * CHUNK, CHUNK)],
                gsem,
            )
            for j in range(n_chunks)
        ]
        for g in gathers:
            g.wait()
        pltpu.sync_copy(rows_v, out_e.at[pl.ds(base, b_per_w)])

    return gather_kernel


def _copy_block(src_ref, dst_ref, tick_ref):
    dst_ref[...] = src_ref[...]
    tick_ref[...] = jnp.zeros_like(tick_ref)


@functools.cache
def _make_wcopy(V, D):
    n_blocks = -(-V // WBLK)
    return pl.pallas_call(
        _copy_block,
        grid=(n_blocks,),
        in_specs=[pl.BlockSpec((D, WBLK), lambda i: (0, i))],
        out_specs=[
            pl.BlockSpec((D, WBLK), lambda i: (0, i)),
            pl.BlockSpec((8, 128), lambda i: (0, 0)),
        ],
        out_shape=[
            jax.ShapeDtypeStruct((D, V), jnp.float32),
            jax.ShapeDtypeStruct((8, 128), jnp.float32),
        ],
    )


def kernel(inputs, embedding_table, nce_weights, nce_biases):
    info = plsc.get_sparse_core_info()
    NW = info.num_cores * info.num_subcores
    idx3 = inputs.reshape(NW, BATCH // NW // CHUNK, CHUNK)
    w_t, tick = _make_wcopy(VOCAB, DIM)(nce_weights.T)
    embed = _make_gather(VOCAB, DIM, BATCH)(idx3, embedding_table, tick)
    return (embed, w_t.T, nce_biases)
